# Initial kernel scaffold; baseline (speedup 1.0000x reference)
import jax
import jax.numpy as jnp
from jax.experimental import pallas as pl


def kernel(hidden_states, gate_w, Wg, Wu, Wd):
    raise NotImplementedError("write your pallas kernel here")


# speedup vs baseline: 1.5968x; 1.5968x over previous
import functools

import jax
import jax.numpy as jnp
from jax import lax
from jax.experimental import pallas as pl
from jax.experimental.pallas import tpu as pltpu
from jax.experimental.pallas import tpu_sc as plsc

T, H, FF, E = 2048, 1024, 2048, 8
BLK = 256
NBLK = 24
PADDED = NBLK * BLK
NBLK_PAD = 128
FFC = 1024
NW = 32
TPW = T // NW
CH = 32


def _router_body(x_ref, gw_ref, pos1_ref, pos2_ref, w1_ref, w2_ref, be_ref):
    x = x_ref[...]
    gw = gw_ref[...]
    logits = lax.dot_general(x, gw, (((1,), (1,)), ((), ())),
                             preferred_element_type=jnp.float32)
    m = jnp.max(logits, axis=1, keepdims=True)
    ex = jnp.exp(logits - m)
    probs = ex / jnp.sum(ex, axis=1, keepdims=True)

    eids = lax.broadcasted_iota(jnp.int32, (T, E), 1)
    m1 = jnp.max(probs, axis=1, keepdims=True)
    e1 = jnp.min(jnp.where(probs >= m1, eids, E), axis=1, keepdims=True)
    one1 = eids == e1
    pm = jnp.where(one1, -1.0, probs)
    m2 = jnp.max(pm, axis=1, keepdims=True)
    e2 = jnp.min(jnp.where(pm >= m2, eids, E), axis=1, keepdims=True)
    one2 = eids == e2

    denom = m1 + m2
    w1 = m1 / denom
    w2 = m2 / denom

    o1 = one1.astype(jnp.float32)
    o2 = one2.astype(jnp.float32)
    c1 = jnp.sum(o1, axis=0, keepdims=True)
    c2 = jnp.sum(o2, axis=0, keepdims=True)
    cnti = (c1 + c2).astype(jnp.int32)
    nblk = (cnti + BLK - 1) // BLK

    tri = (lax.broadcasted_iota(jnp.int32, (E, E), 0)
           < lax.broadcasted_iota(jnp.int32, (E, E), 1)).astype(jnp.float32)
    bstart = lax.dot_general(nblk.astype(jnp.float32), tri,
                             (((1,), (0,)), ((), ())),
                             preferred_element_type=jnp.float32)
    start = BLK * bstart

    def _excl_cumsum0(a):
        acc = a
        sft = 1
        while sft < a.shape[0]:
            z = jnp.zeros((sft, a.shape[1]), a.dtype)
            acc = acc + jnp.concatenate([z, acc[:-sft]], axis=0)
            sft *= 2
        return acc - a

    r1x = _excl_cumsum0(o1)
    r2x = _excl_cumsum0(o2)
    rank1 = jnp.sum(r1x * o1, axis=1, keepdims=True)
    rank2 = (jnp.sum(r2x * o2, axis=1, keepdims=True)
             + jnp.sum(o2 * c1, axis=1, keepdims=True))
    p1 = jnp.sum(o1 * start, axis=1, keepdims=True) + rank1
    p2 = jnp.sum(o2 * start, axis=1, keepdims=True) + rank2

    pos1_ref[...] = p1.astype(jnp.int32)
    pos2_ref[...] = p2.astype(jnp.int32)
    w1_ref[...] = jnp.broadcast_to(w1, (T, 128))
    w2_ref[...] = jnp.broadcast_to(w2, (T, 128))

    bend = start + BLK * nblk.astype(jnp.float32)
    bids = (lax.broadcasted_iota(jnp.int32, (NBLK_PAD, E), 0) * BLK
            ).astype(jnp.float32)
    ge = (bids >= jnp.broadcast_to(bend, (NBLK_PAD, E))).astype(jnp.int32)
    be_ref[...] = jnp.minimum(jnp.sum(ge, axis=1, keepdims=True), E - 1)


def _router(x, gate_w):
    return pl.pallas_call(
        _router_body,
        out_shape=[
            jax.ShapeDtypeStruct((T, 1), jnp.int32),
            jax.ShapeDtypeStruct((T, 1), jnp.int32),
            jax.ShapeDtypeStruct((T, 128), jnp.float32),
            jax.ShapeDtypeStruct((T, 128), jnp.float32),
            jax.ShapeDtypeStruct((NBLK_PAD, 1), jnp.int32),
        ],
    )(x, gate_w)


def _dispatch_body(x_hbm, pos1_hbm, pos2_hbm, w1_hbm, w2_hbm, xs_hbm, ws_hbm,
                   idx1_v, idx2_v, xrows_v, w1_v, w2_v, sem):
    wid = lax.axis_index("s") * 2 + lax.axis_index("c")
    base = wid * TPW
    pltpu.sync_copy(pos1_hbm.at[pl.ds(base, TPW)], idx1_v)
    pltpu.sync_copy(pos2_hbm.at[pl.ds(base, TPW)], idx2_v)
    pltpu.sync_copy(x_hbm.at[pl.ds(base, TPW)], xrows_v)
    pltpu.sync_copy(w1_hbm.at[pl.ds(base, TPW)], w1_v)
    pltpu.sync_copy(w2_hbm.at[pl.ds(base, TPW)], w2_v)
    pltpu.async_copy(xrows_v, xs_hbm.at[idx1_v], sem).wait()
    pltpu.async_copy(xrows_v, xs_hbm.at[idx2_v], sem).wait()
    pltpu.async_copy(w1_v, ws_hbm.at[idx1_v], sem).wait()
    pltpu.async_copy(w2_v, ws_hbm.at[idx2_v], sem).wait()


@functools.lru_cache(maxsize=None)
def _dispatch_kernel():
    mesh = plsc.VectorSubcoreMesh(core_axis_name="c", subcore_axis_name="s")
    return pl.kernel(
        _dispatch_body,
        out_type=[jax.ShapeDtypeStruct((PADDED, H), jnp.float32),
                  jax.ShapeDtypeStruct((PADDED, 128), jnp.float32)],
        mesh=mesh,
        scratch_types=[
            pltpu.VMEM((TPW,), jnp.int32),
            pltpu.VMEM((TPW,), jnp.int32),
            pltpu.VMEM((TPW, H), jnp.float32),
            pltpu.VMEM((TPW, 128), jnp.float32),
            pltpu.VMEM((TPW, 128), jnp.float32),
            pltpu.SemaphoreType.DMA,
        ],
    )


def _up_body(be_ref, xs_ref, wg_ref, wu_ref, h_ref):
    del be_ref
    xsb = xs_ref[...]
    g = lax.dot_general(xsb, wg_ref[0], (((1,), (1,)), ((), ())),
                        preferred_element_type=jnp.float32)
    u = lax.dot_general(xsb, wu_ref[0], (((1,), (1,)), ((), ())),
                        preferred_element_type=jnp.float32)
    h_ref[...] = g * lax.logistic(g) * u


def _up(be, xs, Wg, Wu):
    grid_spec = pltpu.PrefetchScalarGridSpec(
        num_scalar_prefetch=1,
        grid=(FF // FFC, NBLK),
        in_specs=[
            pl.BlockSpec((BLK, H), lambda f, b, be_s: (b, 0)),
            pl.BlockSpec((1, FFC, H), lambda f, b, be_s: (be_s[b], f, 0)),
            pl.BlockSpec((1, FFC, H), lambda f, b, be_s: (be_s[b], f, 0)),
        ],
        out_specs=pl.BlockSpec((BLK, FFC), lambda f, b, be_s: (b, f)),
    )
    return pl.pallas_call(
        _up_body,
        grid_spec=grid_spec,
        out_shape=jax.ShapeDtypeStruct((PADDED, FF), jnp.float32),
    )(be, xs, Wg, Wu)


def _down_body(be_ref, h_ref, wd_ref, ws_ref, ys_ref):
    del be_ref
    o = lax.dot_general(h_ref[...], wd_ref[0], (((1,), (1,)), ((), ())),
                        preferred_element_type=jnp.float32)
    ys_ref[...] = o * ws_ref[:, 0:1]


def _down(be, h, Wd, ws):
    grid_spec = pltpu.PrefetchScalarGridSpec(
        num_scalar_prefetch=1,
        grid=(NBLK,),
        in_specs=[
            pl.BlockSpec((BLK, FF), lambda b, be_s: (b, 0)),
            pl.BlockSpec((1, H, FF), lambda b, be_s: (be_s[b], 0, 0)),
            pl.BlockSpec((BLK, 128), lambda b, be_s: (b, 0)),
        ],
        out_specs=pl.BlockSpec((BLK, H), lambda b, be_s: (b, 0)),
    )
    return pl.pallas_call(
        _down_body,
        grid_spec=grid_spec,
        out_shape=jax.ShapeDtypeStruct((PADDED, H), jnp.float32),
    )(be, h, Wd, ws)


def _combine_body(ys_hbm, pos1_hbm, pos2_hbm, out_hbm, idx1_v, idx2_v, r1_v,
                  r2_v, sem):
    wid = lax.axis_index("s") * 2 + lax.axis_index("c")
    for c in range(TPW // CH):
        base = wid * TPW + c * CH
        pltpu.sync_copy(pos1_hbm.at[pl.ds(base, CH)], idx1_v)
        pltpu.sync_copy(pos2_hbm.at[pl.ds(base, CH)], idx2_v)
        pltpu.async_copy(ys_hbm.at[idx1_v], r1_v, sem).wait()
        pltpu.async_copy(ys_hbm.at[idx2_v], r2_v, sem).wait()

        def row_body(i, _):
            def col_body(j, _):
                sl = pl.ds(j * 16, 16)
                r1_v[i, sl] = r1_v[i, sl] + r2_v[i, sl]
                return 0
            return lax.fori_loop(0, H // 16, col_body, 0)

        lax.fori_loop(0, CH, row_body, 0)
        pltpu.sync_copy(r1_v, out_hbm.at[pl.ds(base, CH)])


@functools.lru_cache(maxsize=None)
def _combine_kernel():
    mesh = plsc.VectorSubcoreMesh(core_axis_name="c", subcore_axis_name="s")
    return pl.kernel(
        _combine_body,
        out_type=jax.ShapeDtypeStruct((T, H), jnp.float32),
        mesh=mesh,
        scratch_types=[
            pltpu.VMEM((CH,), jnp.int32),
            pltpu.VMEM((CH,), jnp.int32),
            pltpu.VMEM((CH, H), jnp.float32),
            pltpu.VMEM((CH, H), jnp.float32),
            pltpu.SemaphoreType.DMA,
        ],
    )


def kernel(hidden_states, gate_w, Wg, Wu, Wd):
    b, s, h = hidden_states.shape
    x = hidden_states.reshape(-1, h)
    pos1c, pos2c, w1x, w2x, bec = _router(x, gate_w)
    pos1 = pos1c.reshape(T)
    pos2 = pos2c.reshape(T)
    be = bec.reshape(-1)[:NBLK]
    xs, ws = _dispatch_kernel()(x, pos1, pos2, w1x, w2x)
    hact = _up(be, xs, Wg, Wu)
    ys = _down(be, hact, Wd, ws)
    out = _combine_kernel()(ys, pos1, pos2)
    return out.reshape(b, s, h)
